# Initial kernel scaffold; baseline (speedup 1.0000x reference)
import jax
import jax.numpy as jnp
from jax.experimental import pallas as pl


def kernel(pcld_input, kpts_pre_input, cpt_pre_input, seg_pre_input, mesh_kpts_input):
    raise NotImplementedError("write your pallas kernel here")


# speedup vs baseline: 5.8214x; 5.8214x over previous
import functools

import jax
import jax.numpy as jnp
from jax.experimental import pallas as pl
from jax.experimental.pallas import tpu as pltpu


def _pose_kernel(offx_ref, offy_ref, offz_ref, pcld_ref, seg_ref, mesh_ref,
                 voted_ref, h_ref, ca_ref, cb_ref, *, n, k):
    ox = offx_ref[0]
    oy = offy_ref[0]
    oz = offz_ref[0]
    pc = pcld_ref[0]
    seg = seg_ref[0]
    mesh = mesh_ref[0]

    score = jnp.sqrt(ox * ox + oy * oy + oz * oz)
    score = score + jnp.where(seg[0:1] >= seg[1:2],
                              jnp.float32(1e10), jnp.float32(0.0))

    tx = ox + pc[0:1]
    ty = oy + pc[1:2]
    tz = oz + pc[2:3]

    iota = jax.lax.broadcasted_iota(jnp.int32, score.shape, 1)
    cx, cy, cz = [], [], []
    for _ in range(k):
        mval = jnp.min(score, axis=1, keepdims=True)
        idxc = jnp.where(score == mval, iota, jnp.int32(n))
        amin = jnp.min(idxc, axis=1, keepdims=True)
        sel = iota == amin
        oh = sel.astype(jnp.float32)
        cx.append(jnp.sum(tx * oh, axis=1, keepdims=True))
        cy.append(jnp.sum(ty * oh, axis=1, keepdims=True))
        cz.append(jnp.sum(tz * oh, axis=1, keepdims=True))
        score = jnp.where(sel, jnp.float32(1e30), score)

    xk = jnp.concatenate(cx, axis=1)
    yk = jnp.concatenate(cy, axis=1)
    zk = jnp.concatenate(cz, axis=1)

    def vote(c):
        m = jnp.mean(c, axis=1, keepdims=True)
        v = jnp.mean((c - m) * (c - m), axis=1, keepdims=True)
        s = jnp.sqrt(v)
        msk = jnp.logical_and(c >= m - s, c <= m + s).astype(c.dtype)
        return jnp.sum(c * msk, axis=1, keepdims=True) / (
            jnp.sum(msk, axis=1, keepdims=True) + jnp.float32(1e-8))

    voted = jnp.concatenate([vote(xk), vote(yk), vote(zk)], axis=1)

    ca = jnp.mean(mesh, axis=0, keepdims=True)
    cb = jnp.mean(voted, axis=0, keepdims=True)
    am = mesh - ca
    bm = voted - cb
    h = jax.lax.dot_general(am, bm, (((0,), (0,)), ((), ())),
                            preferred_element_type=jnp.float32)

    voted_ref[0] = voted
    h_ref[0] = h
    ca_ref[0] = ca
    cb_ref[0] = cb


def kernel(pcld_input, kpts_pre_input, cpt_pre_input, seg_pre_input,
           mesh_kpts_input):
    b, n, _ = pcld_input.shape
    nc = kpts_pre_input.shape[2] + 1
    k = 10

    off = jnp.concatenate([kpts_pre_input, cpt_pre_input], axis=2)
    off_t = jnp.transpose(off, (0, 3, 2, 1))
    offx, offy, offz = off_t[:, 0], off_t[:, 1], off_t[:, 2]
    pcld_t = jnp.transpose(pcld_input, (0, 2, 1))
    seg_t = jnp.transpose(seg_pre_input, (0, 2, 1))

    in_specs = [
        pl.BlockSpec((1, nc, n), lambda i: (i, 0, 0)),
        pl.BlockSpec((1, nc, n), lambda i: (i, 0, 0)),
        pl.BlockSpec((1, nc, n), lambda i: (i, 0, 0)),
        pl.BlockSpec((1, 3, n), lambda i: (i, 0, 0)),
        pl.BlockSpec((1, 2, n), lambda i: (i, 0, 0)),
        pl.BlockSpec((1, nc, 3), lambda i: (i, 0, 0)),
    ]
    out_specs = [
        pl.BlockSpec((1, nc, 3), lambda i: (i, 0, 0)),
        pl.BlockSpec((1, 3, 3), lambda i: (i, 0, 0)),
        pl.BlockSpec((1, 1, 3), lambda i: (i, 0, 0)),
        pl.BlockSpec((1, 1, 3), lambda i: (i, 0, 0)),
    ]
    out_shape = [
        jax.ShapeDtypeStruct((b, nc, 3), jnp.float32),
        jax.ShapeDtypeStruct((b, 3, 3), jnp.float32),
        jax.ShapeDtypeStruct((b, 1, 3), jnp.float32),
        jax.ShapeDtypeStruct((b, 1, 3), jnp.float32),
    ]

    voted, h, ca, cb = pl.pallas_call(
        functools.partial(_pose_kernel, n=n, k=k),
        grid=(b,),
        in_specs=in_specs,
        out_specs=out_specs,
        out_shape=out_shape,
        compiler_params=pltpu.CompilerParams(
            dimension_semantics=("arbitrary",)),
    )(offx, offy, offz, pcld_t, seg_t, mesh_kpts_input)

    u, _, vh = jnp.linalg.svd(h, full_matrices=False)
    v = jnp.swapaxes(vh, -1, -2)
    ut = jnp.swapaxes(u, -1, -2)
    r0 = jnp.matmul(v, ut)
    sign = jnp.sign(jnp.linalg.det(r0))
    colfix = jnp.stack([jnp.ones_like(sign), jnp.ones_like(sign), sign],
                       axis=-1)
    r = jnp.matmul(v * colfix[:, None, :], ut)
    t = jnp.swapaxes(cb, 1, 2) - jnp.matmul(r, jnp.swapaxes(ca, 1, 2))
    return (r, t.reshape(-1, 3), voted)


# speedup vs baseline: 6.2580x; 1.0750x over previous
import functools

import jax
import jax.numpy as jnp
from jax.experimental import pallas as pl
from jax.experimental.pallas import tpu as pltpu


def _pose_kernel(offx_ref, offy_ref, offz_ref, pcld_ref, seg_ref, mesh_ref,
                 voted_ref, h_ref, ca_ref, cb_ref, *, n, k, nc, g):
    ox = offx_ref[...]
    oy = offy_ref[...]
    oz = offz_ref[...]
    pc = pcld_ref[...]
    seg = seg_ref[...]
    mesh = mesh_ref[...]

    score = ox * ox + oy * oy + oz * oz
    tx, ty, tz = [], [], []
    pens = []
    for b in range(g):
        pen = jnp.where(seg[2 * b:2 * b + 1] >= seg[2 * b + 1:2 * b + 2],
                        jnp.float32(1e20), jnp.float32(0.0))
        pens.append(jnp.broadcast_to(pen, (nc, n)))
        tx.append(ox[b * nc:(b + 1) * nc] + pc[3 * b:3 * b + 1])
        ty.append(oy[b * nc:(b + 1) * nc] + pc[3 * b + 1:3 * b + 2])
        tz.append(oz[b * nc:(b + 1) * nc] + pc[3 * b + 2:3 * b + 3])
    score = score + jnp.concatenate(pens, axis=0)
    tx = jnp.concatenate(tx, axis=0)
    ty = jnp.concatenate(ty, axis=0)
    tz = jnp.concatenate(tz, axis=0)

    iota = jax.lax.broadcasted_iota(jnp.int32, score.shape, 1)
    cx, cy, cz = [], [], []
    for _ in range(k):
        mval = jnp.min(score, axis=1, keepdims=True)
        idxc = jnp.where(score == mval, iota, jnp.int32(n))
        amin = jnp.min(idxc, axis=1, keepdims=True)
        sel = iota == amin
        oh = sel.astype(jnp.float32)
        cx.append(jnp.sum(tx * oh, axis=1, keepdims=True))
        cy.append(jnp.sum(ty * oh, axis=1, keepdims=True))
        cz.append(jnp.sum(tz * oh, axis=1, keepdims=True))
        score = jnp.where(sel, jnp.float32(1e30), score)

    xk = jnp.concatenate(cx, axis=1)
    yk = jnp.concatenate(cy, axis=1)
    zk = jnp.concatenate(cz, axis=1)

    def vote(c):
        m = jnp.mean(c, axis=1, keepdims=True)
        v = jnp.mean((c - m) * (c - m), axis=1, keepdims=True)
        s = jnp.sqrt(v)
        msk = jnp.logical_and(c >= m - s, c <= m + s).astype(c.dtype)
        return jnp.sum(c * msk, axis=1, keepdims=True) / (
            jnp.sum(msk, axis=1, keepdims=True) + jnp.float32(1e-8))

    voted = jnp.concatenate([vote(xk), vote(yk), vote(zk)], axis=1)
    voted_ref[...] = voted

    for b in range(g):
        mesh_b = mesh[b * nc:(b + 1) * nc]
        vote_b = voted[b * nc:(b + 1) * nc]
        ca = jnp.mean(mesh_b, axis=0, keepdims=True)
        cb = jnp.mean(vote_b, axis=0, keepdims=True)
        am = mesh_b - ca
        bm = vote_b - cb
        h = jax.lax.dot_general(am, bm, (((0,), (0,)), ((), ())),
                                preferred_element_type=jnp.float32)
        h_ref[b] = h
        ca_ref[b] = ca
        cb_ref[b] = cb


def kernel(pcld_input, kpts_pre_input, cpt_pre_input, seg_pre_input,
           mesh_kpts_input):
    b, n, _ = pcld_input.shape
    nc = kpts_pre_input.shape[2] + 1
    k = 10
    g = 8

    off = jnp.concatenate([kpts_pre_input, cpt_pre_input], axis=2)
    off_t = jnp.transpose(off, (0, 3, 2, 1))
    offx = off_t[:, 0].reshape(b * nc, n)
    offy = off_t[:, 1].reshape(b * nc, n)
    offz = off_t[:, 2].reshape(b * nc, n)
    pcld_t = jnp.transpose(pcld_input, (0, 2, 1)).reshape(b * 3, n)
    seg_t = jnp.transpose(seg_pre_input, (0, 2, 1)).reshape(b * 2, n)
    mesh_r = mesh_kpts_input.reshape(b * nc, 3)

    in_specs = [
        pl.BlockSpec((g * nc, n), lambda i: (i, 0)),
        pl.BlockSpec((g * nc, n), lambda i: (i, 0)),
        pl.BlockSpec((g * nc, n), lambda i: (i, 0)),
        pl.BlockSpec((g * 3, n), lambda i: (i, 0)),
        pl.BlockSpec((g * 2, n), lambda i: (i, 0)),
        pl.BlockSpec((g * nc, 3), lambda i: (i, 0)),
    ]
    out_specs = [
        pl.BlockSpec((g * nc, 3), lambda i: (i, 0)),
        pl.BlockSpec((g, 3, 3), lambda i: (i, 0, 0)),
        pl.BlockSpec((g, 1, 3), lambda i: (i, 0, 0)),
        pl.BlockSpec((g, 1, 3), lambda i: (i, 0, 0)),
    ]
    out_shape = [
        jax.ShapeDtypeStruct((b * nc, 3), jnp.float32),
        jax.ShapeDtypeStruct((b, 3, 3), jnp.float32),
        jax.ShapeDtypeStruct((b, 1, 3), jnp.float32),
        jax.ShapeDtypeStruct((b, 1, 3), jnp.float32),
    ]

    voted, h, ca, cb = pl.pallas_call(
        functools.partial(_pose_kernel, n=n, k=k, nc=nc, g=g),
        grid=(b // g,),
        in_specs=in_specs,
        out_specs=out_specs,
        out_shape=out_shape,
        compiler_params=pltpu.CompilerParams(
            dimension_semantics=("parallel",)),
    )(offx, offy, offz, pcld_t, seg_t, mesh_r)

    u, _, vh = jnp.linalg.svd(h, full_matrices=False)
    v = jnp.swapaxes(vh, -1, -2)
    ut = jnp.swapaxes(u, -1, -2)
    r0 = jnp.matmul(v, ut)
    sign = jnp.sign(jnp.linalg.det(r0))
    colfix = jnp.stack([jnp.ones_like(sign), jnp.ones_like(sign), sign],
                       axis=-1)
    r = jnp.matmul(v * colfix[:, None, :], ut)
    t = jnp.swapaxes(cb, 1, 2) - jnp.matmul(r, jnp.swapaxes(ca, 1, 2))
    return (r, t.reshape(-1, 3), voted.reshape(b, nc, 3))
